# initial kernel scaffold (unmeasured)
import jax
import jax.numpy as jnp
from jax import lax
from jax.experimental import pallas as pl
from jax.experimental.pallas import tpu as pltpu

N_DEV = 4
N_TOK = 2048
D_MODEL = 512
N_EXPERTS = 32
D_FF = 1024
EXP_PER_DEV = N_EXPERTS // N_DEV


def kernel(x, router_W, route_idx, expert_W):
    def body(x_ref, rw_ref, idx_ref, ew_ref, out_ref, comm_ref, send_sems, recv_sems):
        my_pos = lax.axis_index("i")
        left = lax.rem(my_pos + (N_DEV - 1), N_DEV)
        right = lax.rem(my_pos + 1, N_DEV)

        barrier_sem = pltpu.get_barrier_semaphore()
        for nbr in (left, right):
            pl.semaphore_signal(
                barrier_sem, inc=1,
                device_id=(nbr,), device_id_type=pl.DeviceIdType.MESH,
            )
        pl.semaphore_wait(barrier_sem, 2)

        xv = x_ref[:, :]

        scores = jnp.dot(xv, rw_ref[:, :], preferred_element_type=jnp.float32)
        s_max = jnp.max(scores, axis=-1, keepdims=True)
        p = jnp.exp(scores - s_max)
        probs = p / jnp.sum(p, axis=-1, keepdims=True)
        idx = idx_ref[:, :]
        idx0 = idx[:, 0:1]
        idx1 = idx[:, 1:2]
        eids = lax.broadcasted_iota(jnp.int32, (N_TOK, N_EXPERTS), 1)
        g0 = jnp.sum(jnp.where(eids == idx0, probs, 0.0), axis=-1, keepdims=True)
        g1 = jnp.sum(jnp.where(eids == idx1, probs, 0.0), axis=-1, keepdims=True)
        gs = g0 + g1
        w0 = g0 / gs
        w1 = g1 / gs

        acc = jnp.zeros((N_TOK, D_FF), jnp.float32)
        for le in range(EXP_PER_DEV):
            ge = my_pos * EXP_PER_DEV + le
            gate = jnp.where(idx0 == ge, w0, 0.0) + jnp.where(idx1 == ge, w1, 0.0)
            acc = acc + jnp.dot(
                xv * gate, ew_ref[le], preferred_element_type=jnp.float32
            )

        out_ref[:, :] = acc
        comm_ref[0, :, :] = acc

        for h in range(N_DEV - 1):
            send_slot = h % 2
            recv_slot = (h + 1) % 2
            rdma = pltpu.make_async_remote_copy(
                src_ref=comm_ref.at[send_slot],
                dst_ref=comm_ref.at[recv_slot],
                send_sem=send_sems.at[send_slot],
                recv_sem=recv_sems.at[recv_slot],
                device_id=(right,),
                device_id_type=pl.DeviceIdType.MESH,
            )
            rdma.start()
            rdma.wait()
            out_ref[:, :] += comm_ref[recv_slot, :, :]

    return pl.pallas_call(
        body,
        out_shape=jax.ShapeDtypeStruct((N_TOK, D_FF), jnp.float32),
        in_specs=[pl.BlockSpec(memory_space=pltpu.VMEM)] * 4,
        out_specs=pl.BlockSpec(memory_space=pltpu.VMEM),
        scratch_shapes=[
            pltpu.VMEM((2, N_TOK, D_FF), jnp.float32),
            pltpu.SemaphoreType.DMA((2,)),
            pltpu.SemaphoreType.DMA((2,)),
        ],
        compiler_params=pltpu.CompilerParams(
            collective_id=0,
            vmem_limit_bytes=128 * 1024 * 1024,
        ),
    )(x, router_W, route_idx, expert_W)


# baseline (device time: 331402 ns/iter reference)
import jax
import jax.numpy as jnp
from jax import lax
from jax.experimental import pallas as pl
from jax.experimental.pallas import tpu as pltpu

N_DEV = 4
N_TOK = 2048
D_MODEL = 512
N_EXPERTS = 32
D_FF = 1024
EXP_PER_DEV = N_EXPERTS // N_DEV


def kernel(x, router_W, route_idx, expert_W):
    def body(x_ref, rw_ref, idx_ref, ew_ref, out_ref, comm_ref, send_sems, recv_sems):
        my_pos = lax.axis_index("i")
        left = lax.rem(my_pos + (N_DEV - 1), N_DEV)
        right = lax.rem(my_pos + 1, N_DEV)

        barrier_sem = pltpu.get_barrier_semaphore()
        for nbr in (left, right):
            pl.semaphore_signal(
                barrier_sem, inc=1,
                device_id=(nbr,), device_id_type=pl.DeviceIdType.MESH,
            )
        pl.semaphore_wait(barrier_sem, 2)

        xv = x_ref[:, :]

        scores = jnp.dot(xv, rw_ref[:, :], preferred_element_type=jnp.float32)
        s_max = jnp.max(scores, axis=-1, keepdims=True)
        p = jnp.exp(scores - s_max)
        probs = p / jnp.sum(p, axis=-1, keepdims=True)
        idx = idx_ref[:, :]
        idx0 = idx[:, 0:1]
        idx1 = idx[:, 1:2]
        eids = lax.broadcasted_iota(jnp.int32, (N_TOK, N_EXPERTS), 1)
        g0 = jnp.sum(jnp.where(eids == idx0, probs, 0.0), axis=-1, keepdims=True)
        g1 = jnp.sum(jnp.where(eids == idx1, probs, 0.0), axis=-1, keepdims=True)
        gs = g0 + g1
        w0 = g0 / gs
        w1 = g1 / gs

        out_ref[:, :] = jnp.zeros((N_TOK, D_FF), jnp.float32)
        for le in range(EXP_PER_DEV):
            ge = my_pos * EXP_PER_DEV + le
            gate = jnp.where(idx0 == ge, w0, 0.0) + jnp.where(idx1 == ge, w1, 0.0)
            out_ref[:, :] += jnp.dot(
                xv * gate, ew_ref[le], preferred_element_type=jnp.float32
            )

        comm_ref[0, :, :] = out_ref[:, :]

        for h in range(N_DEV - 1):
            send_slot = h % 2
            recv_slot = (h + 1) % 2
            rdma = pltpu.make_async_remote_copy(
                src_ref=comm_ref.at[send_slot],
                dst_ref=comm_ref.at[recv_slot],
                send_sem=send_sems.at[send_slot],
                recv_sem=recv_sems.at[recv_slot],
                device_id=(right,),
                device_id_type=pl.DeviceIdType.MESH,
            )
            rdma.start()
            rdma.wait()
            out_ref[:, :] += comm_ref[recv_slot, :, :]

    return pl.pallas_call(
        body,
        out_shape=jax.ShapeDtypeStruct((N_TOK, D_FF), jnp.float32),
        in_specs=[pl.BlockSpec(memory_space=pltpu.VMEM)] * 4,
        out_specs=pl.BlockSpec(memory_space=pltpu.VMEM),
        scratch_shapes=[
            pltpu.VMEM((2, N_TOK, D_FF), jnp.float32),
            pltpu.SemaphoreType.DMA((2,)),
            pltpu.SemaphoreType.DMA((2,)),
        ],
        compiler_params=pltpu.CompilerParams(
            collective_id=0,
            vmem_limit_bytes=128 * 1024 * 1024,
        ),
    )(x, router_W, route_idx, expert_W)


# device time: 78809 ns/iter; 4.2051x vs baseline; 4.2051x over previous
import jax
import jax.numpy as jnp
from jax import lax
from jax.experimental import pallas as pl
from jax.experimental.pallas import tpu as pltpu

N_DEV = 4
N_TOK = 2048
D_MODEL = 512
N_EXPERTS = 32
D_FF = 1024
EXP_PER_DEV = N_EXPERTS // N_DEV
CHUNK = N_TOK // N_DEV
N_CHAINS = 4
QCOL = D_FF // N_CHAINS
N_STEPS = 2 * (N_DEV - 1)


def kernel(x, router_W, route_idx, expert_W):
    def body(
        x_ref, rw_ref, idx_ref, ew_ref, out_ref,
        comm_ref, stage_ref, ewb_ref, send_sems, recv_sems,
    ):
        my_pos = lax.axis_index("i")
        left = lax.rem(my_pos + (N_DEV - 1), N_DEV)
        right = lax.rem(my_pos + 1, N_DEV)

        barrier_sem = pltpu.get_barrier_semaphore()
        for nbr in (left, right):
            pl.semaphore_signal(
                barrier_sem, inc=1,
                device_id=(nbr,), device_id_type=pl.DeviceIdType.MESH,
            )
        pl.semaphore_wait(barrier_sem, 2)

        def chunk_id(k):
            return lax.rem(my_pos + k + 2 * N_DEV, N_DEV)

        def compute_chunk(c, convert_weights=False):
            row0 = c * CHUNK
            xc = x_ref[pl.ds(row0, CHUNK), :]
            scores = jnp.dot(xc, rw_ref[:, :], preferred_element_type=jnp.float32)
            s_max = jnp.max(scores, axis=-1, keepdims=True)
            p = jnp.exp(scores - s_max)
            probs = p / jnp.sum(p, axis=-1, keepdims=True)
            idx = idx_ref[pl.ds(row0, CHUNK), :]
            i0 = idx[:, 0:1]
            i1 = idx[:, 1:2]
            eids = lax.broadcasted_iota(jnp.int32, (CHUNK, N_EXPERTS), 1)
            g0 = jnp.sum(jnp.where(eids == i0, probs, 0.0), axis=-1, keepdims=True)
            g1 = jnp.sum(jnp.where(eids == i1, probs, 0.0), axis=-1, keepdims=True)
            gs = g0 + g1
            v0 = g0 / gs
            v1 = g1 / gs
            xcb = xc.astype(jnp.bfloat16)
            acc = jnp.zeros((CHUNK, D_FF), jnp.float32)
            for le in range(EXP_PER_DEV):
                if convert_weights:
                    ewb_ref[le, :, :] = ew_ref[le, :, :].astype(jnp.bfloat16)
                ge = my_pos * EXP_PER_DEV + le
                gate = (
                    jnp.where(i0 == ge, v0, 0.0) + jnp.where(i1 == ge, v1, 0.0)
                ).astype(jnp.bfloat16)
                acc = acc + jnp.dot(
                    xcb * gate, ewb_ref[le], preferred_element_type=jnp.float32
                )
            out_ref[pl.ds(row0, CHUNK), :] = acc

        def is_a(r):
            return r < 2

        def send_rdma(r, s, src):
            return pltpu.make_async_remote_copy(
                src_ref=src,
                dst_ref=comm_ref.at[r, s],
                send_sem=send_sems.at[r, s],
                recv_sem=recv_sems.at[r, s],
                device_id=(right if is_a(r) else left,),
                device_id_type=pl.DeviceIdType.MESH,
            )

        def rs_send_chunk(r, s):
            return chunk_id(-s) if is_a(r) else chunk_id(s)

        def rs_fold_chunk(r, s):
            return chunk_id(-s - 1) if is_a(r) else chunk_id(s + 1)

        def fold(r, s):
            c = rs_fold_chunk(r, s)
            cl = r * QCOL
            stage_ref[r, s + 1, :, :] = (
                comm_ref[r, s, :, :].astype(jnp.float32)
                + out_ref[pl.ds(c * CHUNK, CHUNK), cl:cl + QCOL]
            ).astype(jnp.bfloat16)
            rd = send_rdma(r, s + 1, stage_ref.at[r, s + 1])
            rd.start()
            return rd

        compute_chunk(chunk_id(0), convert_weights=True)
        rs = [None] * N_CHAINS
        for r in range(N_CHAINS):
            cl = r * QCOL
            stage_ref[r, 0, :, :] = out_ref[
                pl.ds(chunk_id(0) * CHUNK, CHUNK), cl:cl + QCOL
            ].astype(jnp.bfloat16)
            rs[r] = send_rdma(r, 0, stage_ref.at[r, 0])
            rs[r].start()

        compute_chunk(chunk_id(-1))
        for r in (0, 1):
            rs[r].wait()
            rs[r] = fold(r, 0)
        compute_chunk(chunk_id(1))
        for r in (2, 3):
            rs[r].wait()
            rs[r] = fold(r, 0)
        compute_chunk(chunk_id(2))
        for r in range(N_CHAINS):
            rs[r].wait()
            rs[r] = fold(r, 1)

        ag = [None] * N_CHAINS
        for r in range(N_CHAINS):
            own = chunk_id(1) if is_a(r) else chunk_id(-1)
            cl = r * QCOL
            rs[r].wait()
            out_ref[pl.ds(own * CHUNK, CHUNK), cl:cl + QCOL] += comm_ref[
                r, N_DEV - 2, :, :
            ].astype(jnp.float32)
            stage_ref[r, N_DEV - 1, :, :] = out_ref[
                pl.ds(own * CHUNK, CHUNK), cl:cl + QCOL
            ].astype(jnp.bfloat16)
            ag[r] = send_rdma(r, N_DEV - 1, stage_ref.at[r, N_DEV - 1])
            ag[r].start()

        for s in range(N_DEV - 1):
            for r in range(N_CHAINS):
                ag[r].wait()
                if s < N_DEV - 2:
                    nxt = send_rdma(r, N_DEV + s, comm_ref.at[r, N_DEV - 1 + s])
                    nxt.start()
                else:
                    nxt = None
                c = chunk_id(-s) if is_a(r) else chunk_id(s)
                cl = r * QCOL
                out_ref[pl.ds(c * CHUNK, CHUNK), cl:cl + QCOL] = comm_ref[
                    r, N_DEV - 1 + s, :, :
                ].astype(jnp.float32)
                ag[r] = nxt

    return pl.pallas_call(
        body,
        out_shape=jax.ShapeDtypeStruct((N_TOK, D_FF), jnp.float32),
        in_specs=[pl.BlockSpec(memory_space=pltpu.VMEM)] * 4,
        out_specs=pl.BlockSpec(memory_space=pltpu.VMEM),
        scratch_shapes=[
            pltpu.VMEM((N_CHAINS, N_STEPS, CHUNK, QCOL), jnp.bfloat16),
            pltpu.VMEM((N_CHAINS, N_DEV, CHUNK, QCOL), jnp.bfloat16),
            pltpu.VMEM((EXP_PER_DEV, D_MODEL, D_FF), jnp.bfloat16),
            pltpu.SemaphoreType.DMA((N_CHAINS, N_STEPS)),
            pltpu.SemaphoreType.DMA((N_CHAINS, N_STEPS)),
        ],
        compiler_params=pltpu.CompilerParams(
            collective_id=0,
            vmem_limit_bytes=128 * 1024 * 1024,
        ),
    )(x, router_W, route_idx, expert_W)
